# Initial kernel scaffold; baseline (speedup 1.0000x reference)
import jax
import jax.numpy as jnp
from jax.experimental import pallas as pl


def kernel(adj_logits, adj):
    raise NotImplementedError("write your pallas kernel here")


# speedup vs baseline: 17.9278x; 17.9278x over previous
import jax
import jax.numpy as jnp
from jax.experimental import pallas as pl
from jax.experimental.pallas import tpu as pltpu

_ROUNDS = 8
_SIGN = -2147483648
_MASK31 = 0x7FFFFFFF


def _ukey(z):
    i = jax.lax.bitcast_convert_type(z, jnp.int32)
    key = jnp.where(i >= 0, i, i ^ _MASK31)
    return key ^ _SIGN


def _stats_kernel(z_ref, a_ref, mnmx_ref, ne_ref):
    i = pl.program_id(0)
    j = pl.program_id(1)
    z = z_ref[...]
    a = a_ref[...]
    bm, bn = z.shape
    row = jax.lax.broadcasted_iota(jnp.int32, (bm, bn), 0) + i * bm
    col = jax.lax.broadcasted_iota(jnp.int32, (bm, bn), 1) + j * bn
    up = col > row
    zu = jnp.where(up, z, 0.0)
    bmn = jnp.min(zu)
    bmx = jnp.max(zu)
    bne = jnp.sum(jnp.where(a != 0.0, 1.0, 0.0)).astype(jnp.int32)
    first = (i == 0) & (j == 0)

    @pl.when(first)
    def _():
        mnmx_ref[0, 0] = bmn
        mnmx_ref[0, 1] = bmx
        ne_ref[0, 0] = bne

    @pl.when(jnp.logical_not(first))
    def _():
        mnmx_ref[0, 0] = jnp.minimum(mnmx_ref[0, 0], bmn)
        mnmx_ref[0, 1] = jnp.maximum(mnmx_ref[0, 1], bmx)
        ne_ref[0, 0] = ne_ref[0, 0] + bne


def _make_radix_kernel(n, nb):
    u_tri = n * (n - 1) // 2

    def _radix_kernel(z_ref, a_ref, mnmx_ref, ne_ref, key_ref, kk_ref,
                      hist_ref, state_ref):
        r = pl.program_id(0)
        i = pl.program_id(1)
        j = pl.program_id(2)
        first_blk = (i == 0) & (j == 0)
        last_blk = (i == nb - 1) & (j == nb - 1)
        mn = mnmx_ref[0, 0]

        @pl.when(first_blk & (r == 0))
        def _():
            state_ref[0, 0] = jnp.int32(0)
            state_ref[1, 0] = jnp.int32(0)
            state_ref[0, 1] = jnp.int32(1)
            state_ref[1, 1] = jnp.int32(1)

        @pl.when(first_blk)
        def _():
            for t in range(2):
                for b in range(16):
                    hist_ref[t, b] = jnp.int32(0)

        z = z_ref[...]
        a = a_ref[...]
        bm, bn = z.shape
        row = jax.lax.broadcasted_iota(jnp.int32, (bm, bn), 0) + i * bm
        col = jax.lax.broadcasted_iota(jnp.int32, (bm, bn), 1) + j * bn
        up = col > row
        base = up & (z > mn)
        key = _ukey(z)
        shift = 28 - 4 * r
        nbits = 4 * r
        pmask = (jnp.int32(1) << nbits) - 1
        psh = jnp.minimum(shift + 4, 31)
        hi = (key >> psh) & pmask
        binv = (key >> shift) & 15
        is_edge = a != 0.0
        elig0 = base & is_edge & (hi == state_ref[0, 0])
        elig1 = base & jnp.logical_not(is_edge) & (hi == state_ref[1, 0])
        for t, elig in ((0, elig0), (1, elig1)):
            for b in range(16):
                cnt = jnp.sum(
                    jnp.where(elig & (binv == b), 1.0, 0.0)).astype(jnp.int32)
                hist_ref[t, b] = hist_ref[t, b] + cnt

        @pl.when(last_blk)
        def _():
            ne = ne_ref[0, 0]
            nc = (ne * 3) // 20
            mn_neg = mn < 0.0
            m_counts = (ne // 2, u_tri - ne // 2 + n)
            zero_bin = jnp.where(r == 0, 8, 0)
            zp_sh = jnp.maximum(4 * r - 4, 0)
            zero_prefix = jnp.where(r == 0, 0, jnp.int32(8) << zp_sh)
            for t in range(2):
                m_t = jnp.where(mn_neg, jnp.int32(m_counts[t]), 0)
                prefix = state_ref[t, 0]
                inj_on = prefix == zero_prefix
                h = []
                for b in range(16):
                    inj = jnp.where(inj_on & (zero_bin == b), m_t, 0)
                    h.append(hist_ref[t, b] + inj)
                total = h[0]
                for b in range(1, 16):
                    total = total + h[b]
                k = jnp.minimum(total, nc)
                if t == 0:
                    fresh_rank = k
                else:
                    fresh_rank = total - k + 1
                rank = jnp.where(r == 0, jnp.maximum(fresh_rank, 1),
                                 state_ref[t, 1])

                @pl.when(r == 0)
                def _(k=k, t=t):
                    kk_ref[0, t] = k

                cum = jnp.int32(0)
                chosen = jnp.int32(0)
                basec = jnp.int32(0)
                found = jnp.int32(0) == jnp.int32(1)
                for b in range(16):
                    cum2 = cum + h[b]
                    hit = jnp.logical_and(jnp.logical_not(found), cum2 >= rank)
                    chosen = jnp.where(hit, b, chosen)
                    basec = jnp.where(hit, cum, basec)
                    found = jnp.logical_or(found, hit)
                    cum = cum2
                newpref = (prefix << 4) | chosen
                state_ref[t, 0] = newpref
                state_ref[t, 1] = rank - basec

                @pl.when(r == _ROUNDS - 1)
                def _(newpref=newpref, t=t):
                    key_ref[0, t] = newpref

    return _radix_kernel


def _apply_kernel(z_ref, zt_ref, a_ref, mnmx_ref, zk_ref, kk_ref, o_ref):
    i = pl.program_id(0)
    j = pl.program_id(1)
    mn = mnmx_ref[0, 0]
    denom = mnmx_ref[0, 1] - mn
    z = z_ref[...]
    zt = zt_ref[...]
    a = a_ref[...]
    bm, bn = z.shape
    row = jax.lax.broadcasted_iota(jnp.int32, (bm, bn), 0) + i * bm
    col = jax.lax.broadcasted_iota(jnp.int32, (bm, bn), 1) + j * bn

    def vnorm(x):
        v = (jnp.full((8, 128), x, jnp.float32) - mn) / denom
        return jnp.max(v)

    c = vnorm(jnp.float32(0.0))
    th_rm = vnorm(zk_ref[0, 0])
    th_add = vnorm(zk_ref[0, 1])
    n_rm = kk_ref[0, 0]
    n_add = kk_ref[0, 1]

    pz = (z - mn) / denom
    pzt = (zt - mn) / denom
    p_ij = jnp.where(col > row, pz, c)
    p_ji = jnp.where(col < row, pzt, c)

    ainv = 1.0 - a
    mrm = p_ij * a
    mrm_t = p_ji * a
    madd = p_ij * ainv
    madd_t = p_ji * ainv

    krm = jnp.where((mrm > 0.0) & (mrm <= th_rm), 1.0, 0.0)
    krm_t = jnp.where((mrm_t > 0.0) & (mrm_t <= th_rm), 1.0, 0.0)
    kadd = jnp.where((madd > 0.0) & (madd >= th_add), 1.0, 0.0)
    kadd_t = jnp.where((madd_t > 0.0) & (madd_t >= th_add), 1.0, 0.0)

    rm_c = jnp.where(n_rm > 0, krm + krm_t, mrm)
    add_c = jnp.where(n_add > 0, kadd + kadd_t, madd)
    o_ref[...] = (a - rm_c) + add_c


def kernel(adj_logits, adj):
    n = adj_logits.shape[0]
    b = 512 if n % 512 == 0 else n
    nb = n // b

    blk = lambda im: pl.BlockSpec((b, b), im)
    smem = pl.BlockSpec(memory_space=pltpu.SMEM)

    mnmx, ne = pl.pallas_call(
        _stats_kernel,
        grid=(nb, nb),
        in_specs=[blk(lambda i, j: (i, j)), blk(lambda i, j: (i, j))],
        out_specs=[smem, smem],
        out_shape=[jax.ShapeDtypeStruct((1, 2), jnp.float32),
                   jax.ShapeDtypeStruct((1, 1), jnp.int32)],
    )(adj_logits, adj)

    keys, kk = pl.pallas_call(
        _make_radix_kernel(n, nb),
        grid=(_ROUNDS, nb, nb),
        in_specs=[blk(lambda r, i, j: (i, j)), blk(lambda r, i, j: (i, j)),
                  smem, smem],
        out_specs=[smem, smem],
        out_shape=[jax.ShapeDtypeStruct((1, 2), jnp.int32),
                   jax.ShapeDtypeStruct((1, 2), jnp.int32)],
        scratch_shapes=[pltpu.SMEM((2, 16), jnp.int32),
                        pltpu.SMEM((2, 2), jnp.int32)],
    )(adj_logits, adj, mnmx, ne)

    skey = keys ^ _SIGN
    iv = jnp.where(skey >= 0, skey, skey ^ _MASK31)
    zk = jax.lax.bitcast_convert_type(iv, jnp.float32)

    out = pl.pallas_call(
        _apply_kernel,
        grid=(nb, nb),
        in_specs=[blk(lambda i, j: (i, j)), blk(lambda i, j: (i, j)),
                  blk(lambda i, j: (i, j)), smem, smem, smem],
        out_specs=blk(lambda i, j: (i, j)),
        out_shape=jax.ShapeDtypeStruct((n, n), jnp.float32),
    )(adj_logits, adj_logits.T, adj, mnmx, zk, kk)
    return out


# speedup vs baseline: 36.4763x; 2.0346x over previous
import jax
import jax.numpy as jnp
from jax.experimental import pallas as pl
from jax.experimental.pallas import tpu as pltpu

_ROUNDS = 8
_SIGN = -2147483648
_MASK31 = 0x7FFFFFFF


def _ukey(z):
    i = jax.lax.bitcast_convert_type(z, jnp.int32)
    key = jnp.where(i >= 0, i, i ^ _MASK31)
    return key ^ _SIGN


def _make_stats_kernel(nt):
    def _stats_kernel(bi_ref, bj_ref, z_ref, a_ref, mnmx_ref, neu_ref):
        t = pl.program_id(0)
        i = bi_ref[t]
        j = bj_ref[t]
        z = z_ref[...]
        a = a_ref[...]
        bm, bn = z.shape
        row = jax.lax.broadcasted_iota(jnp.int32, (bm, bn), 0) + i * bm
        col = jax.lax.broadcasted_iota(jnp.int32, (bm, bn), 1) + j * bn
        up = col > row
        zu = jnp.where(up, z, 0.0)
        bmn = jnp.min(zu)
        bmx = jnp.max(zu)
        bne = jnp.sum(jnp.where(up & (a != 0.0), 1.0, 0.0)).astype(jnp.int32)

        @pl.when(t == 0)
        def _():
            mnmx_ref[0, 0] = jnp.minimum(bmn, 0.0)
            mnmx_ref[0, 1] = jnp.maximum(bmx, 0.0)
            neu_ref[0, 0] = bne

        @pl.when(t != 0)
        def _():
            mnmx_ref[0, 0] = jnp.minimum(mnmx_ref[0, 0], bmn)
            mnmx_ref[0, 1] = jnp.maximum(mnmx_ref[0, 1], bmx)
            neu_ref[0, 0] = neu_ref[0, 0] + bne

    return _stats_kernel


def _make_radix_kernel(n, nt):
    u_tri = n * (n - 1) // 2

    def _radix_kernel(bi_ref, bj_ref, z_ref, a_ref, mnmx_ref, neu_ref,
                      key_ref, kk_ref, hist_ref, state_ref):
        r = pl.program_id(0)
        t = pl.program_id(1)
        i = bi_ref[t]
        j = bj_ref[t]
        first_blk = t == 0
        last_blk = t == nt - 1
        mn = mnmx_ref[0, 0]

        @pl.when(first_blk & (r == 0))
        def _():
            state_ref[0, 0] = jnp.int32(0)
            state_ref[1, 0] = jnp.int32(0)
            state_ref[0, 1] = jnp.int32(1)
            state_ref[1, 1] = jnp.int32(1)

        @pl.when(first_blk)
        def _():
            for tt in range(2):
                for b in range(16):
                    hist_ref[tt, b] = jnp.int32(0)

        z = z_ref[...]
        a = a_ref[...]
        bm, bn = z.shape
        row = jax.lax.broadcasted_iota(jnp.int32, (bm, bn), 0) + i * bm
        col = jax.lax.broadcasted_iota(jnp.int32, (bm, bn), 1) + j * bn
        up = col > row
        base = up & (z > mn)
        key = _ukey(z)
        shift = 28 - 4 * r
        nbits = 4 * r
        pmask = (jnp.int32(1) << nbits) - 1
        psh = jnp.minimum(shift + 4, 31)
        hi = (key >> psh) & pmask
        binv = (key >> shift) & 15
        is_edge = a != 0.0
        elig0 = base & is_edge & (hi == state_ref[0, 0])
        elig1 = base & jnp.logical_not(is_edge) & (hi == state_ref[1, 0])
        for tt, elig in ((0, elig0), (1, elig1)):
            binm = jnp.where(elig, binv, 16)
            for b in range(16):
                cnt = jnp.sum(jnp.where(binm == b, 1.0, 0.0)).astype(jnp.int32)
                hist_ref[tt, b] = hist_ref[tt, b] + cnt

        @pl.when(last_blk)
        def _():
            ne = neu_ref[0, 0] * 2
            nc = (ne * 3) // 20
            mn_neg = mn < 0.0
            m_counts = (ne // 2, u_tri - ne // 2 + n)
            zero_bin = jnp.where(r == 0, 8, 0)
            zp_sh = jnp.maximum(4 * r - 4, 0)
            zero_prefix = jnp.where(r == 0, 0, jnp.int32(8) << zp_sh)
            for tt in range(2):
                m_t = jnp.where(mn_neg, jnp.int32(m_counts[tt]), 0)
                prefix = state_ref[tt, 0]
                inj_on = prefix == zero_prefix
                h = []
                for b in range(16):
                    inj = jnp.where(inj_on & (zero_bin == b), m_t, 0)
                    h.append(hist_ref[tt, b] + inj)
                total = h[0]
                for b in range(1, 16):
                    total = total + h[b]
                k = jnp.minimum(total, nc)
                if tt == 0:
                    fresh_rank = k
                else:
                    fresh_rank = total - k + 1
                rank = jnp.where(r == 0, jnp.maximum(fresh_rank, 1),
                                 state_ref[tt, 1])

                @pl.when(r == 0)
                def _(k=k, tt=tt):
                    kk_ref[0, tt] = k

                cum = jnp.int32(0)
                chosen = jnp.int32(0)
                basec = jnp.int32(0)
                found = jnp.int32(0) == jnp.int32(1)
                for b in range(16):
                    cum2 = cum + h[b]
                    hit = jnp.logical_and(jnp.logical_not(found), cum2 >= rank)
                    chosen = jnp.where(hit, b, chosen)
                    basec = jnp.where(hit, cum, basec)
                    found = jnp.logical_or(found, hit)
                    cum = cum2
                newpref = (prefix << 4) | chosen
                state_ref[tt, 0] = newpref
                state_ref[tt, 1] = rank - basec

                @pl.when(r == _ROUNDS - 1)
                def _(newpref=newpref, tt=tt):
                    key_ref[0, tt] = newpref

    return _radix_kernel


def _apply_kernel(z_ref, zt_ref, a_ref, mnmx_ref, zk_ref, kk_ref, o_ref):
    i = pl.program_id(0)
    j = pl.program_id(1)
    mn = mnmx_ref[0, 0]
    denom = mnmx_ref[0, 1] - mn
    z = z_ref[...]
    zt = zt_ref[...]
    a = a_ref[...]
    bm, bn = z.shape
    row = jax.lax.broadcasted_iota(jnp.int32, (bm, bn), 0) + i * bm
    col = jax.lax.broadcasted_iota(jnp.int32, (bm, bn), 1) + j * bn

    def vnorm(x):
        v = (jnp.full((8, 128), x, jnp.float32) - mn) / denom
        return jnp.max(v)

    c = vnorm(jnp.float32(0.0))
    th_rm = vnorm(zk_ref[0, 0])
    th_add = vnorm(zk_ref[0, 1])
    n_rm = kk_ref[0, 0]
    n_add = kk_ref[0, 1]

    pz = (z - mn) / denom
    pzt = (zt - mn) / denom
    p_ij = jnp.where(col > row, pz, c)
    p_ji = jnp.where(col < row, pzt, c)

    ainv = 1.0 - a
    mrm = p_ij * a
    mrm_t = p_ji * a
    madd = p_ij * ainv
    madd_t = p_ji * ainv

    krm = jnp.where((mrm > 0.0) & (mrm <= th_rm), 1.0, 0.0)
    krm_t = jnp.where((mrm_t > 0.0) & (mrm_t <= th_rm), 1.0, 0.0)
    kadd = jnp.where((madd > 0.0) & (madd >= th_add), 1.0, 0.0)
    kadd_t = jnp.where((madd_t > 0.0) & (madd_t >= th_add), 1.0, 0.0)

    rm_c = jnp.where(n_rm > 0, krm + krm_t, mrm)
    add_c = jnp.where(n_add > 0, kadd + kadd_t, madd)
    o_ref[...] = (a - rm_c) + add_c


def kernel(adj_logits, adj):
    n = adj_logits.shape[0]
    b = 512 if n % 512 == 0 else n
    nb = n // b
    pairs = [(i, j) for i in range(nb) for j in range(i, nb)]
    nt = len(pairs)
    bi = jnp.asarray([p[0] for p in pairs], jnp.int32)
    bj = jnp.asarray([p[1] for p in pairs], jnp.int32)

    blk = lambda im: pl.BlockSpec((b, b), im)
    smem = pl.BlockSpec(memory_space=pltpu.SMEM)

    mnmx, neu = pl.pallas_call(
        _make_stats_kernel(nt),
        grid_spec=pltpu.PrefetchScalarGridSpec(
            num_scalar_prefetch=2,
            grid=(nt,),
            in_specs=[blk(lambda t, bi, bj: (bi[t], bj[t])),
                      blk(lambda t, bi, bj: (bi[t], bj[t]))],
            out_specs=[smem, smem],
        ),
        out_shape=[jax.ShapeDtypeStruct((1, 2), jnp.float32),
                   jax.ShapeDtypeStruct((1, 1), jnp.int32)],
    )(bi, bj, adj_logits, adj)

    keys, kk = pl.pallas_call(
        _make_radix_kernel(n, nt),
        grid_spec=pltpu.PrefetchScalarGridSpec(
            num_scalar_prefetch=2,
            grid=(_ROUNDS, nt),
            in_specs=[blk(lambda r, t, bi, bj: (bi[t], bj[t])),
                      blk(lambda r, t, bi, bj: (bi[t], bj[t])),
                      smem, smem],
            out_specs=[smem, smem],
            scratch_shapes=[pltpu.SMEM((2, 16), jnp.int32),
                            pltpu.SMEM((2, 2), jnp.int32)],
        ),
        out_shape=[jax.ShapeDtypeStruct((1, 2), jnp.int32),
                   jax.ShapeDtypeStruct((1, 2), jnp.int32)],
    )(bi, bj, adj_logits, adj, mnmx, neu)

    skey = keys ^ _SIGN
    iv = jnp.where(skey >= 0, skey, skey ^ _MASK31)
    zk = jax.lax.bitcast_convert_type(iv, jnp.float32)

    out = pl.pallas_call(
        _apply_kernel,
        grid=(nb, nb),
        in_specs=[blk(lambda i, j: (i, j)), blk(lambda i, j: (i, j)),
                  blk(lambda i, j: (i, j)), smem, smem, smem],
        out_specs=blk(lambda i, j: (i, j)),
        out_shape=jax.ShapeDtypeStruct((n, n), jnp.float32),
        compiler_params=pltpu.CompilerParams(
            dimension_semantics=("parallel", "parallel")),
    )(adj_logits, adj_logits.T, adj, mnmx, zk, kk)
    return out


# speedup vs baseline: 56.2092x; 1.5410x over previous
import jax
import jax.numpy as jnp
from jax.experimental import pallas as pl
from jax.experimental.pallas import tpu as pltpu

_ROUNDS = 8
_SIGN = -2147483648
_MASK31 = 0x7FFFFFFF


def _ukey(z):
    i = jax.lax.bitcast_convert_type(z, jnp.int32)
    key = jnp.where(i >= 0, i, i ^ _MASK31)
    return key ^ _SIGN


def _make_stats_kernel(nt):
    def _stats_kernel(bi_ref, bj_ref, z_ref, a_ref, mnmx_ref, neu_ref):
        t = pl.program_id(0)
        i = bi_ref[t]
        j = bj_ref[t]
        z = z_ref[...]
        a = a_ref[...]
        bm, bn = z.shape
        row = jax.lax.broadcasted_iota(jnp.int32, (bm, bn), 0) + i * bm
        col = jax.lax.broadcasted_iota(jnp.int32, (bm, bn), 1) + j * bn
        up = col > row
        zu = jnp.where(up, z, 0.0)
        bmn = jnp.min(zu)
        bmx = jnp.max(zu)
        bne = jnp.sum(jnp.where(up & (a != 0.0), 1.0, 0.0)).astype(jnp.int32)

        @pl.when(t == 0)
        def _():
            mnmx_ref[0, 0] = jnp.minimum(bmn, 0.0)
            mnmx_ref[0, 1] = jnp.maximum(bmx, 0.0)
            neu_ref[0, 0] = bne

        @pl.when(t != 0)
        def _():
            mnmx_ref[0, 0] = jnp.minimum(mnmx_ref[0, 0], bmn)
            mnmx_ref[0, 1] = jnp.maximum(mnmx_ref[0, 1], bmx)
            neu_ref[0, 0] = neu_ref[0, 0] + bne

    return _stats_kernel


def _make_radix_kernel(n, nt):
    u_tri = n * (n - 1) // 2

    def _radix_kernel(bi_ref, bj_ref, z_ref, a_ref, mnmx_ref, neu_ref,
                      key_ref, kk_ref, hist_ref, state_ref):
        r = pl.program_id(0)
        t = pl.program_id(1)
        i = bi_ref[t]
        j = bj_ref[t]
        first_blk = t == 0
        last_blk = t == nt - 1
        mn = mnmx_ref[0, 0]

        @pl.when(first_blk & (r == 0))
        def _():
            state_ref[0, 0] = jnp.int32(0)
            state_ref[1, 0] = jnp.int32(0)
            state_ref[0, 1] = jnp.int32(1)
            state_ref[1, 1] = jnp.int32(1)

        @pl.when(first_blk)
        def _():
            for tt in range(2):
                for b in range(16):
                    hist_ref[tt, b] = jnp.int32(0)

        z = z_ref[...]
        a = a_ref[...]
        bm, bn = z.shape
        row = jax.lax.broadcasted_iota(jnp.int32, (bm, bn), 0) + i * bm
        col = jax.lax.broadcasted_iota(jnp.int32, (bm, bn), 1) + j * bn
        up = col > row
        base = up & (z > mn)
        key = _ukey(z)
        shift = 28 - 4 * r
        nbits = 4 * r
        pmask = (jnp.int32(1) << nbits) - 1
        psh = jnp.minimum(shift + 4, 31)
        hi = (key >> psh) & pmask
        binv = (key >> shift) & 15
        is_edge = a != 0.0
        elig0 = base & is_edge & (hi == state_ref[0, 0])
        elig1 = base & jnp.logical_not(is_edge) & (hi == state_ref[1, 0])
        lo2 = binv & 3
        hi2 = binv >> 2
        powv = jnp.int32(1) << (lo2 << 3)
        rsh = lambda x: x.reshape(bm // 8, 8, bn)
        pm0 = rsh(jnp.where(elig0, powv, 0))
        pm1 = rsh(jnp.where(elig1, powv, 0))
        hi2R = rsh(hi2)
        for h in range(4):
            eqh = hi2R == h
            for tt, pm in ((0, pm0), (1, pm1)):
                part = jnp.sum(jnp.where(eqh, pm, 0), axis=0)
                for l in range(4):
                    cnt = jnp.sum((part >> (8 * l)) & 255)
                    b = h * 4 + l
                    hist_ref[tt, b] = hist_ref[tt, b] + cnt

        @pl.when(last_blk)
        def _():
            ne = neu_ref[0, 0] * 2
            nc = (ne * 3) // 20
            mn_neg = mn < 0.0
            m_counts = (ne // 2, u_tri - ne // 2 + n)
            zero_bin = jnp.where(r == 0, 8, 0)
            zp_sh = jnp.maximum(4 * r - 4, 0)
            zero_prefix = jnp.where(r == 0, 0, jnp.int32(8) << zp_sh)
            for tt in range(2):
                m_t = jnp.where(mn_neg, jnp.int32(m_counts[tt]), 0)
                prefix = state_ref[tt, 0]
                inj_on = prefix == zero_prefix
                h = []
                for b in range(16):
                    inj = jnp.where(inj_on & (zero_bin == b), m_t, 0)
                    h.append(hist_ref[tt, b] + inj)
                total = h[0]
                for b in range(1, 16):
                    total = total + h[b]
                k = jnp.minimum(total, nc)
                if tt == 0:
                    fresh_rank = k
                else:
                    fresh_rank = total - k + 1
                rank = jnp.where(r == 0, jnp.maximum(fresh_rank, 1),
                                 state_ref[tt, 1])

                @pl.when(r == 0)
                def _(k=k, tt=tt):
                    kk_ref[0, tt] = k

                cum = jnp.int32(0)
                chosen = jnp.int32(0)
                basec = jnp.int32(0)
                found = jnp.int32(0) == jnp.int32(1)
                for b in range(16):
                    cum2 = cum + h[b]
                    hit = jnp.logical_and(jnp.logical_not(found), cum2 >= rank)
                    chosen = jnp.where(hit, b, chosen)
                    basec = jnp.where(hit, cum, basec)
                    found = jnp.logical_or(found, hit)
                    cum = cum2
                newpref = (prefix << 4) | chosen
                state_ref[tt, 0] = newpref
                state_ref[tt, 1] = rank - basec

                @pl.when(r == _ROUNDS - 1)
                def _(newpref=newpref, tt=tt):
                    key_ref[0, tt] = newpref

    return _radix_kernel


def _apply_kernel(z_ref, zt_ref, a_ref, mnmx_ref, zk_ref, kk_ref, o_ref):
    i = pl.program_id(0)
    j = pl.program_id(1)
    mn = mnmx_ref[0, 0]
    denom = mnmx_ref[0, 1] - mn
    z = z_ref[...]
    zt = zt_ref[...]
    a = a_ref[...]
    bm, bn = z.shape
    row = jax.lax.broadcasted_iota(jnp.int32, (bm, bn), 0) + i * bm
    col = jax.lax.broadcasted_iota(jnp.int32, (bm, bn), 1) + j * bn

    def vnorm(x):
        v = (jnp.full((8, 128), x, jnp.float32) - mn) / denom
        return jnp.max(v)

    c = vnorm(jnp.float32(0.0))
    th_rm = vnorm(zk_ref[0, 0])
    th_add = vnorm(zk_ref[0, 1])
    n_rm = kk_ref[0, 0]
    n_add = kk_ref[0, 1]

    pz = (z - mn) / denom
    pzt = (zt - mn) / denom
    p_ij = jnp.where(col > row, pz, c)
    p_ji = jnp.where(col < row, pzt, c)

    ainv = 1.0 - a
    mrm = p_ij * a
    mrm_t = p_ji * a
    madd = p_ij * ainv
    madd_t = p_ji * ainv

    krm = jnp.where((mrm > 0.0) & (mrm <= th_rm), 1.0, 0.0)
    krm_t = jnp.where((mrm_t > 0.0) & (mrm_t <= th_rm), 1.0, 0.0)
    kadd = jnp.where((madd > 0.0) & (madd >= th_add), 1.0, 0.0)
    kadd_t = jnp.where((madd_t > 0.0) & (madd_t >= th_add), 1.0, 0.0)

    rm_c = jnp.where(n_rm > 0, krm + krm_t, mrm)
    add_c = jnp.where(n_add > 0, kadd + kadd_t, madd)
    o_ref[...] = (a - rm_c) + add_c


def kernel(adj_logits, adj):
    n = adj_logits.shape[0]
    b = 512 if n % 512 == 0 else n
    nb = n // b
    pairs = [(i, j) for i in range(nb) for j in range(i, nb)]
    nt = len(pairs)
    bi = jnp.asarray([p[0] for p in pairs], jnp.int32)
    bj = jnp.asarray([p[1] for p in pairs], jnp.int32)

    blk = lambda im: pl.BlockSpec((b, b), im)
    smem = pl.BlockSpec(memory_space=pltpu.SMEM)

    mnmx, neu = pl.pallas_call(
        _make_stats_kernel(nt),
        grid_spec=pltpu.PrefetchScalarGridSpec(
            num_scalar_prefetch=2,
            grid=(nt,),
            in_specs=[blk(lambda t, bi, bj: (bi[t], bj[t])),
                      blk(lambda t, bi, bj: (bi[t], bj[t]))],
            out_specs=[smem, smem],
        ),
        out_shape=[jax.ShapeDtypeStruct((1, 2), jnp.float32),
                   jax.ShapeDtypeStruct((1, 1), jnp.int32)],
    )(bi, bj, adj_logits, adj)

    keys, kk = pl.pallas_call(
        _make_radix_kernel(n, nt),
        grid_spec=pltpu.PrefetchScalarGridSpec(
            num_scalar_prefetch=2,
            grid=(_ROUNDS, nt),
            in_specs=[blk(lambda r, t, bi, bj: (bi[t], bj[t])),
                      blk(lambda r, t, bi, bj: (bi[t], bj[t])),
                      smem, smem],
            out_specs=[smem, smem],
            scratch_shapes=[pltpu.SMEM((2, 16), jnp.int32),
                            pltpu.SMEM((2, 2), jnp.int32)],
        ),
        out_shape=[jax.ShapeDtypeStruct((1, 2), jnp.int32),
                   jax.ShapeDtypeStruct((1, 2), jnp.int32)],
    )(bi, bj, adj_logits, adj, mnmx, neu)

    skey = keys ^ _SIGN
    iv = jnp.where(skey >= 0, skey, skey ^ _MASK31)
    zk = jax.lax.bitcast_convert_type(iv, jnp.float32)

    out = pl.pallas_call(
        _apply_kernel,
        grid=(nb, nb),
        in_specs=[blk(lambda i, j: (i, j)), blk(lambda i, j: (i, j)),
                  blk(lambda i, j: (i, j)), smem, smem, smem],
        out_specs=blk(lambda i, j: (i, j)),
        out_shape=jax.ShapeDtypeStruct((n, n), jnp.float32),
        compiler_params=pltpu.CompilerParams(
            dimension_semantics=("parallel", "parallel")),
    )(adj_logits, adj_logits.T, adj, mnmx, zk, kk)
    return out


# speedup vs baseline: 64.9964x; 1.1563x over previous
import jax
import jax.numpy as jnp
from jax.experimental import pallas as pl
from jax.experimental.pallas import tpu as pltpu

_ROUNDS = 8
_SIGN = -2147483648
_MASK31 = 0x7FFFFFFF


def _ukey(z):
    i = jax.lax.bitcast_convert_type(z, jnp.int32)
    key = jnp.where(i >= 0, i, i ^ _MASK31)
    return key ^ _SIGN


def _make_stats_kernel(nt):
    def _stats_kernel(bi_ref, bj_ref, z_ref, a_ref, mnmx_ref, neu_ref):
        t = pl.program_id(0)
        i = bi_ref[t]
        j = bj_ref[t]
        z = z_ref[...]
        a = a_ref[...]
        bm, bn = z.shape
        row = jax.lax.broadcasted_iota(jnp.int32, (bm, bn), 0) + i * bm
        col = jax.lax.broadcasted_iota(jnp.int32, (bm, bn), 1) + j * bn
        up = col > row
        zu = jnp.where(up, z, 0.0)
        bmn = jnp.min(zu)
        bmx = jnp.max(zu)
        bne = jnp.sum(jnp.where(up & (a != 0.0), 1.0, 0.0)).astype(jnp.int32)

        @pl.when(t == 0)
        def _():
            mnmx_ref[0, 0] = jnp.minimum(bmn, 0.0)
            mnmx_ref[0, 1] = jnp.maximum(bmx, 0.0)
            neu_ref[0, 0] = bne

        @pl.when(t != 0)
        def _():
            mnmx_ref[0, 0] = jnp.minimum(mnmx_ref[0, 0], bmn)
            mnmx_ref[0, 1] = jnp.maximum(mnmx_ref[0, 1], bmx)
            neu_ref[0, 0] = neu_ref[0, 0] + bne

    return _stats_kernel


def _make_radix_kernel(n, nt):
    u_tri = n * (n - 1) // 2

    def _radix_kernel(bi_ref, bj_ref, z_ref, a_ref, mnmx_ref, neu_ref,
                      key_ref, kk_ref, hist_ref, state_ref, bcnt_ref):
        r = pl.program_id(0)
        t = pl.program_id(1)
        i = bi_ref[t]
        j = bj_ref[t]
        first_blk = t == 0
        last_blk = t == nt - 1
        mn = mnmx_ref[0, 0]

        @pl.when(first_blk & (r == 0))
        def _():
            state_ref[0, 0] = jnp.int32(0)
            state_ref[1, 0] = jnp.int32(0)
            state_ref[0, 1] = jnp.int32(1)
            state_ref[1, 1] = jnp.int32(1)

        @pl.when(first_blk)
        def _():
            for tt in range(2):
                for b in range(16):
                    hist_ref[tt, b] = jnp.int32(0)

        ch0 = state_ref[0, 0] & 15
        ch1 = state_ref[1, 0] & 15
        act0 = (r == 0) | (bcnt_ref[0, t, ch0] != 0)
        act1 = (r == 0) | (bcnt_ref[1, t, ch1] != 0)

        @pl.when(act0 | act1)
        def _():
            z = z_ref[...]
            a = a_ref[...]
            bm, bn = z.shape
            row = jax.lax.broadcasted_iota(jnp.int32, (bm, bn), 0)
            col = jax.lax.broadcasted_iota(jnp.int32, (bm, bn), 1)
            up = (col > row) | (j > i)
            base = up & (z > mn)
            key = _ukey(z)
            shift = 28 - 4 * r
            nbits = 4 * r
            pmask = (jnp.int32(1) << nbits) - 1
            psh = jnp.minimum(shift + 4, 31)
            hi = (key >> psh) & pmask
            binv = (key >> shift) & 15
            is_edge = a != 0.0
            lo2 = binv & 3
            hi2 = binv >> 2
            powv = jnp.int32(1) << (lo2 << 3)
            rsh = lambda x: x.reshape(bm // 8, 8, bn)
            hi2R = rsh(hi2)
            eqh = [hi2R == h for h in range(4)]
            for tt, act, cls in ((0, act0, is_edge),
                                 (1, act1, jnp.logical_not(is_edge))):
                @pl.when(act)
                def _(tt=tt, cls=cls):
                    elig = base & cls & (hi == state_ref[tt, 0])
                    pm = rsh(jnp.where(elig, powv, 0))
                    for h in range(4):
                        part = jnp.sum(jnp.where(eqh[h], pm, 0), axis=0)
                        for l in range(4):
                            cnt = jnp.sum((part >> (8 * l)) & 255)
                            b = h * 4 + l
                            hist_ref[tt, b] = hist_ref[tt, b] + cnt
                            bcnt_ref[tt, t, b] = cnt

        for tt, act in ((0, act0), (1, act1)):
            @pl.when(jnp.logical_not(act))
            def _(tt=tt):
                for b in range(16):
                    bcnt_ref[tt, t, b] = jnp.int32(0)

        @pl.when(last_blk)
        def _():
            ne = neu_ref[0, 0] * 2
            nc = (ne * 3) // 20
            mn_neg = mn < 0.0
            m_counts = (ne // 2, u_tri - ne // 2 + n)
            zero_bin = jnp.where(r == 0, 8, 0)
            zp_sh = jnp.maximum(4 * r - 4, 0)
            zero_prefix = jnp.where(r == 0, 0, jnp.int32(8) << zp_sh)
            for tt in range(2):
                m_t = jnp.where(mn_neg, jnp.int32(m_counts[tt]), 0)
                prefix = state_ref[tt, 0]
                inj_on = prefix == zero_prefix
                h = []
                for b in range(16):
                    inj = jnp.where(inj_on & (zero_bin == b), m_t, 0)
                    h.append(hist_ref[tt, b] + inj)
                total = h[0]
                for b in range(1, 16):
                    total = total + h[b]
                k = jnp.minimum(total, nc)
                if tt == 0:
                    fresh_rank = k
                else:
                    fresh_rank = total - k + 1
                rank = jnp.where(r == 0, jnp.maximum(fresh_rank, 1),
                                 state_ref[tt, 1])

                @pl.when(r == 0)
                def _(k=k, tt=tt):
                    kk_ref[0, tt] = k

                cum = jnp.int32(0)
                chosen = jnp.int32(0)
                basec = jnp.int32(0)
                found = jnp.int32(0) == jnp.int32(1)
                for b in range(16):
                    cum2 = cum + h[b]
                    hit = jnp.logical_and(jnp.logical_not(found), cum2 >= rank)
                    chosen = jnp.where(hit, b, chosen)
                    basec = jnp.where(hit, cum, basec)
                    found = jnp.logical_or(found, hit)
                    cum = cum2
                newpref = (prefix << 4) | chosen
                state_ref[tt, 0] = newpref
                state_ref[tt, 1] = rank - basec

                @pl.when(r == _ROUNDS - 1)
                def _(newpref=newpref, tt=tt):
                    key_ref[0, tt] = newpref

    return _radix_kernel


def _apply_kernel(z_ref, zt_ref, a_ref, mnmx_ref, zk_ref, kk_ref, o_ref):
    i = pl.program_id(0)
    j = pl.program_id(1)
    mn = mnmx_ref[0, 0]
    denom = mnmx_ref[0, 1] - mn
    z = z_ref[...]
    zt = zt_ref[...]
    a = a_ref[...]
    bm, bn = z.shape
    row = jax.lax.broadcasted_iota(jnp.int32, (bm, bn), 0) + i * bm
    col = jax.lax.broadcasted_iota(jnp.int32, (bm, bn), 1) + j * bn

    def vnorm(x):
        v = (jnp.full((8, 128), x, jnp.float32) - mn) / denom
        return jnp.max(v)

    c = vnorm(jnp.float32(0.0))
    th_rm = vnorm(zk_ref[0, 0])
    th_add = vnorm(zk_ref[0, 1])
    n_rm = kk_ref[0, 0]
    n_add = kk_ref[0, 1]

    pz = (z - mn) / denom
    pzt = (zt - mn) / denom
    p_ij = jnp.where(col > row, pz, c)
    p_ji = jnp.where(col < row, pzt, c)

    ainv = 1.0 - a
    mrm = p_ij * a
    mrm_t = p_ji * a
    madd = p_ij * ainv
    madd_t = p_ji * ainv

    krm = jnp.where((mrm > 0.0) & (mrm <= th_rm), 1.0, 0.0)
    krm_t = jnp.where((mrm_t > 0.0) & (mrm_t <= th_rm), 1.0, 0.0)
    kadd = jnp.where((madd > 0.0) & (madd >= th_add), 1.0, 0.0)
    kadd_t = jnp.where((madd_t > 0.0) & (madd_t >= th_add), 1.0, 0.0)

    rm_c = jnp.where(n_rm > 0, krm + krm_t, mrm)
    add_c = jnp.where(n_add > 0, kadd + kadd_t, madd)
    o_ref[...] = (a - rm_c) + add_c


def kernel(adj_logits, adj):
    n = adj_logits.shape[0]
    b = 512 if n % 512 == 0 else n
    nb = n // b
    pairs = [(i, j) for i in range(nb) for j in range(i, nb)]
    nt = len(pairs)
    bi = jnp.asarray([p[0] for p in pairs], jnp.int32)
    bj = jnp.asarray([p[1] for p in pairs], jnp.int32)

    blk = lambda im: pl.BlockSpec((b, b), im)
    smem = pl.BlockSpec(memory_space=pltpu.SMEM)

    mnmx, neu = pl.pallas_call(
        _make_stats_kernel(nt),
        grid_spec=pltpu.PrefetchScalarGridSpec(
            num_scalar_prefetch=2,
            grid=(nt,),
            in_specs=[blk(lambda t, bi, bj: (bi[t], bj[t])),
                      blk(lambda t, bi, bj: (bi[t], bj[t]))],
            out_specs=[smem, smem],
        ),
        out_shape=[jax.ShapeDtypeStruct((1, 2), jnp.float32),
                   jax.ShapeDtypeStruct((1, 1), jnp.int32)],
    )(bi, bj, adj_logits, adj)

    keys, kk = pl.pallas_call(
        _make_radix_kernel(n, nt),
        grid_spec=pltpu.PrefetchScalarGridSpec(
            num_scalar_prefetch=2,
            grid=(_ROUNDS, nt),
            in_specs=[blk(lambda r, t, bi, bj: (bi[t], bj[t])),
                      blk(lambda r, t, bi, bj: (bi[t], bj[t])),
                      smem, smem],
            out_specs=[smem, smem],
            scratch_shapes=[pltpu.SMEM((2, 16), jnp.int32),
                            pltpu.SMEM((2, 2), jnp.int32),
                            pltpu.SMEM((2, nt, 16), jnp.int32)],
        ),
        out_shape=[jax.ShapeDtypeStruct((1, 2), jnp.int32),
                   jax.ShapeDtypeStruct((1, 2), jnp.int32)],
    )(bi, bj, adj_logits, adj, mnmx, neu)

    skey = keys ^ _SIGN
    iv = jnp.where(skey >= 0, skey, skey ^ _MASK31)
    zk = jax.lax.bitcast_convert_type(iv, jnp.float32)

    out = pl.pallas_call(
        _apply_kernel,
        grid=(nb, nb),
        in_specs=[blk(lambda i, j: (i, j)), blk(lambda i, j: (i, j)),
                  blk(lambda i, j: (i, j)), smem, smem, smem],
        out_specs=blk(lambda i, j: (i, j)),
        out_shape=jax.ShapeDtypeStruct((n, n), jnp.float32),
        compiler_params=pltpu.CompilerParams(
            dimension_semantics=("parallel", "parallel")),
    )(adj_logits, adj_logits.T, adj, mnmx, zk, kk)
    return out


# speedup vs baseline: 71.6353x; 1.1021x over previous
import jax
import jax.numpy as jnp
from jax.experimental import pallas as pl
from jax.experimental.pallas import tpu as pltpu

_ROUNDS = 8
_SIGN = -2147483648
_MASK31 = 0x7FFFFFFF


def _ukey(z):
    i = jax.lax.bitcast_convert_type(z, jnp.int32)
    key = jnp.where(i >= 0, i, i ^ _MASK31)
    return key ^ _SIGN


def _make_stats_kernel(nt):
    def _stats_kernel(bi_ref, bj_ref, z_ref, a_ref, mnmx_ref, neu_ref):
        t = pl.program_id(0)
        i = bi_ref[t]
        j = bj_ref[t]
        z = z_ref[...]
        a = a_ref[...]
        bm, bn = z.shape
        row = jax.lax.broadcasted_iota(jnp.int32, (bm, bn), 0) + i * bm
        col = jax.lax.broadcasted_iota(jnp.int32, (bm, bn), 1) + j * bn
        up = col > row
        zu = jnp.where(up, z, 0.0)
        bmn = jnp.min(zu)
        bmx = jnp.max(zu)
        bne = jnp.sum(jnp.where(up & (a != 0.0), 1.0, 0.0)).astype(jnp.int32)

        @pl.when(t == 0)
        def _():
            mnmx_ref[0, 0] = jnp.minimum(bmn, 0.0)
            mnmx_ref[0, 1] = jnp.maximum(bmx, 0.0)
            neu_ref[0, 0] = bne

        @pl.when(t != 0)
        def _():
            mnmx_ref[0, 0] = jnp.minimum(mnmx_ref[0, 0], bmn)
            mnmx_ref[0, 1] = jnp.maximum(mnmx_ref[0, 1], bmx)
            neu_ref[0, 0] = neu_ref[0, 0] + bne

    return _stats_kernel


def _make_radix_kernel(n, nt):
    u_tri = n * (n - 1) // 2

    def _radix_kernel(bi_ref, bj_ref, z_ref, a_ref, mnmx_ref, neu_ref,
                      key_ref, kk_ref, hist_ref, state_ref, bcnt_ref):
        r = pl.program_id(0)
        t = pl.program_id(1)
        i = bi_ref[t]
        j = bj_ref[t]
        first_blk = t == 0
        last_blk = t == nt - 1
        mn = mnmx_ref[0, 0]

        @pl.when(first_blk & (r == 0))
        def _():
            state_ref[0, 0] = jnp.int32(0)
            state_ref[1, 0] = jnp.int32(0)
            state_ref[0, 1] = jnp.int32(1)
            state_ref[1, 1] = jnp.int32(1)

        @pl.when(first_blk)
        def _():
            for tt in range(2):
                for b in range(16):
                    hist_ref[tt, b] = jnp.int32(0)

        ch0 = state_ref[0, 0] & 15
        ch1 = state_ref[1, 0] & 15
        act0 = (r == 0) | (bcnt_ref[0, t, ch0] != 0)
        act1 = (r == 0) | (bcnt_ref[1, t, ch1] != 0)

        @pl.when(act0 | act1)
        def _():
            z = z_ref[...]
            a = a_ref[...]
            bm, bn = z.shape
            row = jax.lax.broadcasted_iota(jnp.int32, (bm, bn), 0)
            col = jax.lax.broadcasted_iota(jnp.int32, (bm, bn), 1)
            up = (col > row) | (j > i)
            base = up & (z > mn)
            key = _ukey(z)
            shift = 28 - 4 * r
            nbits = 4 * r
            pmask = (jnp.int32(1) << nbits) - 1
            psh = jnp.minimum(shift + 4, 31)
            hi = (key >> psh) & pmask
            binv = (key >> shift) & 15
            is_edge = a != 0.0
            lo2 = binv & 3
            hi2 = binv >> 2
            powv = jnp.int32(1) << (lo2 << 3)
            rsh = lambda x: x.reshape(bm // 8, 8, bn)
            hi2R = rsh(hi2)
            eqh = [hi2R == h for h in range(4)]
            for tt, act, cls in ((0, act0, is_edge),
                                 (1, act1, jnp.logical_not(is_edge))):
                @pl.when(act)
                def _(tt=tt, cls=cls):
                    elig = base & cls & (hi == state_ref[tt, 0])
                    pm = rsh(jnp.where(elig, powv, 0))
                    for h in range(4):
                        part = jnp.sum(jnp.where(eqh[h], pm, 0), axis=0)
                        for l in range(4):
                            cnt = jnp.sum((part >> (8 * l)) & 255)
                            b = h * 4 + l
                            hist_ref[tt, b] = hist_ref[tt, b] + cnt
                            bcnt_ref[tt, t, b] = cnt

        for tt, act in ((0, act0), (1, act1)):
            @pl.when(jnp.logical_not(act))
            def _(tt=tt):
                for b in range(16):
                    bcnt_ref[tt, t, b] = jnp.int32(0)

        @pl.when(last_blk)
        def _():
            ne = neu_ref[0, 0] * 2
            nc = (ne * 3) // 20
            mn_neg = mn < 0.0
            m_counts = (ne // 2, u_tri - ne // 2 + n)
            zero_bin = jnp.where(r == 0, 8, 0)
            zp_sh = jnp.maximum(4 * r - 4, 0)
            zero_prefix = jnp.where(r == 0, 0, jnp.int32(8) << zp_sh)
            for tt in range(2):
                m_t = jnp.where(mn_neg, jnp.int32(m_counts[tt]), 0)
                prefix = state_ref[tt, 0]
                inj_on = prefix == zero_prefix
                h = []
                for b in range(16):
                    inj = jnp.where(inj_on & (zero_bin == b), m_t, 0)
                    h.append(hist_ref[tt, b] + inj)
                total = h[0]
                for b in range(1, 16):
                    total = total + h[b]
                k = jnp.minimum(total, nc)
                if tt == 0:
                    fresh_rank = k
                else:
                    fresh_rank = total - k + 1
                rank = jnp.where(r == 0, jnp.maximum(fresh_rank, 1),
                                 state_ref[tt, 1])

                @pl.when(r == 0)
                def _(k=k, tt=tt):
                    kk_ref[0, tt] = k

                cum = jnp.int32(0)
                chosen = jnp.int32(0)
                basec = jnp.int32(0)
                found = jnp.int32(0) == jnp.int32(1)
                for b in range(16):
                    cum2 = cum + h[b]
                    hit = jnp.logical_and(jnp.logical_not(found), cum2 >= rank)
                    chosen = jnp.where(hit, b, chosen)
                    basec = jnp.where(hit, cum, basec)
                    found = jnp.logical_or(found, hit)
                    cum = cum2
                newpref = (prefix << 4) | chosen
                state_ref[tt, 0] = newpref
                state_ref[tt, 1] = rank - basec

                @pl.when(r == _ROUNDS - 1)
                def _(newpref=newpref, tt=tt):
                    key_ref[0, tt] = newpref

    return _radix_kernel


def _apply_kernel(z_ref, zt_ref, a_ref, mnmx_ref, zk_ref, kk_ref, o_ref):
    i = pl.program_id(0)
    j = pl.program_id(1)
    mn = mnmx_ref[0, 0]
    denom = mnmx_ref[0, 1] - mn
    z = z_ref[...]
    zt = zt_ref[...].T
    a = a_ref[...]
    bm, bn = z.shape
    row = jax.lax.broadcasted_iota(jnp.int32, (bm, bn), 0) + i * bm
    col = jax.lax.broadcasted_iota(jnp.int32, (bm, bn), 1) + j * bn

    def vnorm(x):
        v = (jnp.full((8, 128), x, jnp.float32) - mn) / denom
        return jnp.max(v)

    c = vnorm(jnp.float32(0.0))
    th_rm = vnorm(zk_ref[0, 0])
    th_add = vnorm(zk_ref[0, 1])
    n_rm = kk_ref[0, 0]
    n_add = kk_ref[0, 1]

    pz = (z - mn) / denom
    pzt = (zt - mn) / denom
    p_ij = jnp.where(col > row, pz, c)
    p_ji = jnp.where(col < row, pzt, c)

    ainv = 1.0 - a
    mrm = p_ij * a
    mrm_t = p_ji * a
    madd = p_ij * ainv
    madd_t = p_ji * ainv

    krm = jnp.where((mrm > 0.0) & (mrm <= th_rm), 1.0, 0.0)
    krm_t = jnp.where((mrm_t > 0.0) & (mrm_t <= th_rm), 1.0, 0.0)
    kadd = jnp.where((madd > 0.0) & (madd >= th_add), 1.0, 0.0)
    kadd_t = jnp.where((madd_t > 0.0) & (madd_t >= th_add), 1.0, 0.0)

    rm_c = jnp.where(n_rm > 0, krm + krm_t, mrm)
    add_c = jnp.where(n_add > 0, kadd + kadd_t, madd)
    o_ref[...] = (a - rm_c) + add_c


def kernel(adj_logits, adj):
    n = adj_logits.shape[0]
    b = 512 if n % 512 == 0 else n
    nb = n // b
    pairs = [(i, j) for i in range(nb) for j in range(i, nb)]
    nt = len(pairs)
    bi = jnp.asarray([p[0] for p in pairs], jnp.int32)
    bj = jnp.asarray([p[1] for p in pairs], jnp.int32)

    blk = lambda im: pl.BlockSpec((b, b), im)
    smem = pl.BlockSpec(memory_space=pltpu.SMEM)

    mnmx, neu = pl.pallas_call(
        _make_stats_kernel(nt),
        grid_spec=pltpu.PrefetchScalarGridSpec(
            num_scalar_prefetch=2,
            grid=(nt,),
            in_specs=[blk(lambda t, bi, bj: (bi[t], bj[t])),
                      blk(lambda t, bi, bj: (bi[t], bj[t]))],
            out_specs=[smem, smem],
        ),
        out_shape=[jax.ShapeDtypeStruct((1, 2), jnp.float32),
                   jax.ShapeDtypeStruct((1, 1), jnp.int32)],
    )(bi, bj, adj_logits, adj)

    keys, kk = pl.pallas_call(
        _make_radix_kernel(n, nt),
        grid_spec=pltpu.PrefetchScalarGridSpec(
            num_scalar_prefetch=2,
            grid=(_ROUNDS, nt),
            in_specs=[blk(lambda r, t, bi, bj: (bi[t], bj[t])),
                      blk(lambda r, t, bi, bj: (bi[t], bj[t])),
                      smem, smem],
            out_specs=[smem, smem],
            scratch_shapes=[pltpu.SMEM((2, 16), jnp.int32),
                            pltpu.SMEM((2, 2), jnp.int32),
                            pltpu.SMEM((2, nt, 16), jnp.int32)],
        ),
        out_shape=[jax.ShapeDtypeStruct((1, 2), jnp.int32),
                   jax.ShapeDtypeStruct((1, 2), jnp.int32)],
    )(bi, bj, adj_logits, adj, mnmx, neu)

    skey = keys ^ _SIGN
    iv = jnp.where(skey >= 0, skey, skey ^ _MASK31)
    zk = jax.lax.bitcast_convert_type(iv, jnp.float32)

    out = pl.pallas_call(
        _apply_kernel,
        grid=(nb, nb),
        in_specs=[blk(lambda i, j: (i, j)), blk(lambda i, j: (j, i)),
                  blk(lambda i, j: (i, j)), smem, smem, smem],
        out_specs=blk(lambda i, j: (i, j)),
        out_shape=jax.ShapeDtypeStruct((n, n), jnp.float32),
        compiler_params=pltpu.CompilerParams(
            dimension_semantics=("parallel", "parallel")),
    )(adj_logits, adj_logits, adj, mnmx, zk, kk)
    return out
